# Initial kernel scaffold; baseline (speedup 1.0000x reference)
import jax
import jax.numpy as jnp
from jax.experimental import pallas as pl


def kernel(features, w0, w1):
    raise NotImplementedError("write your pallas kernel here")


# speedup vs baseline: 11.3109x; 11.3109x over previous
import jax
import jax.numpy as jnp
from jax.experimental import pallas as pl
from jax.experimental.pallas import tpu as pltpu

_KK = 31
_R = 256
_CT = 512


def _emb_kernel(f_ref, w0_ref, w1_ref, o_ref):
    h = jnp.maximum(f_ref[...] * w0_ref[...], 0.0) * w1_ref[...]
    nrm = jnp.sqrt(jnp.sum(h * h, axis=1, keepdims=True))
    o_ref[...] = h / jnp.maximum(nrm, 1e-12)


def _strip_kernel(a_ref, bT_ref, o_ref, strip_ref, *, nt):
    a = a_ref[...]
    for j in range(nt):
        tile = jnp.dot(a, bT_ref[:, pl.ds(j * _CT, _CT)],
                       preferred_element_type=jnp.float32)
        strip_ref[:, pl.ds(j * _CT, _CT)] = jnp.maximum(tile, 0.0)

    def pop(_, t):
        acc = jnp.full((a.shape[0], _CT), -1.0, jnp.float32)
        for j in range(nt):
            tile = strip_ref[:, pl.ds(j * _CT, _CT)]
            acc = jnp.maximum(acc, jnp.where(tile < t, tile, -1.0))
        return jnp.max(acc, axis=1, keepdims=True)

    t0 = jnp.full((a.shape[0], 1), jnp.inf, dtype=jnp.float32)
    t = jax.lax.fori_loop(0, _KK, pop, t0)

    for j in range(nt):
        tile = strip_ref[:, pl.ds(j * _CT, _CT)]
        o_ref[:, pl.ds(j * _CT, _CT)] = jnp.where(tile >= t, tile, 0.0)


def kernel(features, w0, w1):
    n, d = features.shape
    npad = -(-n // _CT) * _CT
    nt = npad // _CT

    f = features
    if npad != n:
        f = jnp.pad(features, ((0, npad - n), (0, 0)))

    rb = 1024 if npad % 1024 == 0 else _R
    emb = pl.pallas_call(
        _emb_kernel,
        grid=(npad // rb,),
        in_specs=[
            pl.BlockSpec((rb, d), lambda i: (i, 0)),
            pl.BlockSpec((1, d), lambda i: (0, 0)),
            pl.BlockSpec((1, d), lambda i: (0, 0)),
        ],
        out_specs=pl.BlockSpec((rb, d), lambda i: (i, 0)),
        out_shape=jax.ShapeDtypeStruct((npad, d), jnp.float32),
    )(f, w0.reshape(1, d), w1.reshape(1, d))

    embT = emb.T

    import functools
    out = pl.pallas_call(
        functools.partial(_strip_kernel, nt=nt),
        grid=(npad // _R,),
        in_specs=[
            pl.BlockSpec((_R, d), lambda i: (i, 0)),
            pl.BlockSpec((d, npad), lambda i: (0, 0)),
        ],
        out_specs=pl.BlockSpec((_R, npad), lambda i: (i, 0)),
        out_shape=jax.ShapeDtypeStruct((npad, npad), jnp.float32),
        scratch_shapes=[pltpu.VMEM((_R, npad), jnp.float32)],
    )(emb, embT)

    return out[:n, :n]


# speedup vs baseline: 13.1396x; 1.1617x over previous
import jax
import jax.numpy as jnp
from jax.experimental import pallas as pl
from jax.experimental.pallas import tpu as pltpu

_KK = 31
_R = 256
_CT = 512


def _emb_kernel(f_ref, w0_ref, w1_ref, o_ref):
    h = jnp.maximum(f_ref[...] * w0_ref[...], 0.0) * w1_ref[...]
    nrm = jnp.sqrt(jnp.sum(h * h, axis=1, keepdims=True))
    o_ref[...] = h / jnp.maximum(nrm, 1e-12)


def _strip_kernel(a_ref, bT_ref, o_ref, strip_ref, *, nt, n):
    a = a_ref[...]
    for j in range(nt):
        tile = jnp.dot(a, bT_ref[:, pl.ds(j * _CT, _CT)],
                       preferred_element_type=jnp.float32)
        strip_ref[:, pl.ds(j * _CT, _CT)] = jnp.maximum(tile, 0.0)

    def pop(_, t):
        acc = jnp.full((a.shape[0], _CT), -1.0, jnp.float32)
        for j in range(nt):
            tile = strip_ref[:, pl.ds(j * _CT, _CT)]
            acc = jnp.maximum(acc, jnp.where(tile < t, tile, -1.0))
        return jnp.max(acc, axis=1, keepdims=True)

    t0 = jnp.full((a.shape[0], 1), jnp.inf, dtype=jnp.float32)
    t = jax.lax.fori_loop(0, _KK, pop, t0)

    for j in range(nt):
        start = j * _CT
        if start >= n:
            break
        w = min(_CT, n - start)
        tile = strip_ref[:, pl.ds(start, _CT)]
        masked = jnp.where(tile >= t, tile, 0.0)
        o_ref[:, pl.ds(start, w)] = masked[:, :w]


def kernel(features, w0, w1):
    n, d = features.shape
    npad = -(-n // _CT) * _CT
    nt = npad // _CT

    f = features
    if npad != n:
        f = jnp.pad(features, ((0, npad - n), (0, 0)))

    rb = 1024 if npad % 1024 == 0 else _R
    emb = pl.pallas_call(
        _emb_kernel,
        grid=(npad // rb,),
        in_specs=[
            pl.BlockSpec((rb, d), lambda i: (i, 0)),
            pl.BlockSpec((1, d), lambda i: (0, 0)),
            pl.BlockSpec((1, d), lambda i: (0, 0)),
        ],
        out_specs=pl.BlockSpec((rb, d), lambda i: (i, 0)),
        out_shape=jax.ShapeDtypeStruct((npad, d), jnp.float32),
    )(f, w0.reshape(1, d), w1.reshape(1, d))

    embT = emb.T

    import functools
    out = pl.pallas_call(
        functools.partial(_strip_kernel, nt=nt, n=n),
        grid=(npad // _R,),
        in_specs=[
            pl.BlockSpec((_R, d), lambda i: (i, 0)),
            pl.BlockSpec((d, npad), lambda i: (0, 0)),
        ],
        out_specs=pl.BlockSpec((_R, n), lambda i: (i, 0)),
        out_shape=jax.ShapeDtypeStruct((n, n), jnp.float32),
        scratch_shapes=[pltpu.VMEM((_R, npad), jnp.float32)],
        compiler_params=pltpu.CompilerParams(
            dimension_semantics=("parallel",)),
    )(emb, embT)

    return out


# speedup vs baseline: 27.2637x; 2.0749x over previous
import jax
import jax.numpy as jnp
from jax.experimental import pallas as pl
from jax.experimental.pallas import tpu as pltpu

_KK = 31
_R = 256
_CT = 512


def _emb_kernel(f_ref, w0_ref, w1_ref, o_ref):
    h = jnp.maximum(f_ref[...] * w0_ref[...], 0.0) * w1_ref[...]
    nrm = jnp.sqrt(jnp.sum(h * h, axis=1, keepdims=True))
    o_ref[...] = h / jnp.maximum(nrm, 1e-12)


_P = 10


def _strip_kernel(a_ref, bT_ref, o_ref, strip_ref, *, nt, n):
    a = a_ref[...]
    r = a.shape[0]
    slots = [jnp.full((r, 128), -1.0, jnp.float32) for _ in range(_P)]
    for j in range(nt):
        tile = jnp.maximum(
            jnp.dot(a, bT_ref[:, pl.ds(j * _CT, _CT)],
                    preferred_element_type=jnp.float32), 0.0)
        strip_ref[:, pl.ds(j * _CT, _CT)] = tile
        for q in range(_CT // 128):
            cur = tile[:, q * 128:(q + 1) * 128]
            for s in range(_P):
                hi = jnp.maximum(slots[s], cur)
                cur = jnp.minimum(slots[s], cur)
                slots[s] = hi

    cand = jnp.concatenate(slots, axis=1)

    def pop(_, t):
        masked = jnp.where(cand < t, cand, -1.0)
        return jnp.max(masked, axis=1, keepdims=True)

    t0 = jnp.full((r, 1), jnp.inf, dtype=jnp.float32)
    t = jax.lax.fori_loop(0, _KK, pop, t0)

    for j in range(nt):
        start = j * _CT
        if start >= n:
            break
        w = min(_CT, n - start)
        tile = strip_ref[:, pl.ds(start, _CT)]
        masked = jnp.where(tile >= t, tile, 0.0)
        o_ref[:, pl.ds(start, w)] = masked[:, :w]


def kernel(features, w0, w1):
    n, d = features.shape
    npad = -(-n // _CT) * _CT
    nt = npad // _CT

    f = features
    if npad != n:
        f = jnp.pad(features, ((0, npad - n), (0, 0)))

    rb = 1024 if npad % 1024 == 0 else _R
    emb = pl.pallas_call(
        _emb_kernel,
        grid=(npad // rb,),
        in_specs=[
            pl.BlockSpec((rb, d), lambda i: (i, 0)),
            pl.BlockSpec((1, d), lambda i: (0, 0)),
            pl.BlockSpec((1, d), lambda i: (0, 0)),
        ],
        out_specs=pl.BlockSpec((rb, d), lambda i: (i, 0)),
        out_shape=jax.ShapeDtypeStruct((npad, d), jnp.float32),
    )(f, w0.reshape(1, d), w1.reshape(1, d))

    embT = emb.T

    import functools
    out = pl.pallas_call(
        functools.partial(_strip_kernel, nt=nt, n=n),
        grid=(npad // _R,),
        in_specs=[
            pl.BlockSpec((_R, d), lambda i: (i, 0)),
            pl.BlockSpec((d, npad), lambda i: (0, 0)),
        ],
        out_specs=pl.BlockSpec((_R, n), lambda i: (i, 0)),
        out_shape=jax.ShapeDtypeStruct((n, n), jnp.float32),
        scratch_shapes=[pltpu.VMEM((_R, npad), jnp.float32)],
        compiler_params=pltpu.CompilerParams(
            dimension_semantics=("parallel",)),
    )(emb, embT)

    return out


# speedup vs baseline: 52.9320x; 1.9415x over previous
import jax
import jax.numpy as jnp
from jax.experimental import pallas as pl
from jax.experimental.pallas import tpu as pltpu

_KK = 31
_R = 256
_CT = 512


def _emb_kernel(f_ref, w0_ref, w1_ref, o_ref):
    h = jnp.maximum(f_ref[...] * w0_ref[...], 0.0) * w1_ref[...]
    nrm = jnp.sqrt(jnp.sum(h * h, axis=1, keepdims=True))
    o_ref[...] = h / jnp.maximum(nrm, 1e-12)


_P = 4
_G = 2


def _strip_kernel(a_ref, bT_ref, o_ref, strip_ref, *, nt, n):
    a = a_ref[...]
    r = a.shape[0]
    nq = _CT // 128
    slots = [[jnp.full((r, 128), -1.0, jnp.float32) for _ in range(_P)]
             for _ in range(_G)]
    for j in range(nt):
        tile = jnp.maximum(
            jnp.dot(a, bT_ref[:, pl.ds(j * _CT, _CT)],
                    preferred_element_type=jnp.float32), 0.0)
        strip_ref[:, pl.ds(j * _CT, _CT)] = tile
        for q in range(nq):
            g = q * _G // nq
            cur = tile[:, q * 128:(q + 1) * 128]
            for s in range(_P):
                hi = jnp.maximum(slots[g][s], cur)
                cur = jnp.minimum(slots[g][s], cur)
                slots[g][s] = hi

    cand = jnp.concatenate(slots[0] + slots[1], axis=1)

    t = jnp.full((r, 1), jnp.inf, dtype=jnp.float32)
    for _ in range(_KK):
        masked = jnp.where(cand < t, cand, -1.0)
        t = jnp.max(masked, axis=1, keepdims=True)

    for j in range(nt):
        start = j * _CT
        if start >= n:
            break
        w = min(_CT, n - start)
        tile = strip_ref[:, pl.ds(start, _CT)]
        masked = jnp.where(tile >= t, tile, 0.0)
        o_ref[:, pl.ds(start, w)] = masked[:, :w]


def kernel(features, w0, w1):
    n, d = features.shape
    npad = -(-n // _CT) * _CT
    nt = npad // _CT

    f = features
    if npad != n:
        f = jnp.pad(features, ((0, npad - n), (0, 0)))

    rb = 1024 if npad % 1024 == 0 else _R
    emb = pl.pallas_call(
        _emb_kernel,
        grid=(npad // rb,),
        in_specs=[
            pl.BlockSpec((rb, d), lambda i: (i, 0)),
            pl.BlockSpec((1, d), lambda i: (0, 0)),
            pl.BlockSpec((1, d), lambda i: (0, 0)),
        ],
        out_specs=pl.BlockSpec((rb, d), lambda i: (i, 0)),
        out_shape=jax.ShapeDtypeStruct((npad, d), jnp.float32),
    )(f, w0.reshape(1, d), w1.reshape(1, d))

    embT = emb.T

    import functools
    out = pl.pallas_call(
        functools.partial(_strip_kernel, nt=nt, n=n),
        grid=(npad // _R,),
        in_specs=[
            pl.BlockSpec((_R, d), lambda i: (i, 0)),
            pl.BlockSpec((d, npad), lambda i: (0, 0)),
        ],
        out_specs=pl.BlockSpec((_R, n), lambda i: (i, 0)),
        out_shape=jax.ShapeDtypeStruct((n, n), jnp.float32),
        scratch_shapes=[pltpu.VMEM((_R, npad), jnp.float32)],
        compiler_params=pltpu.CompilerParams(
            dimension_semantics=("parallel",)),
    )(emb, embT)

    return out


# speedup vs baseline: 54.0376x; 1.0209x over previous
import jax
import jax.numpy as jnp
from jax.experimental import pallas as pl
from jax.experimental.pallas import tpu as pltpu

_KK = 31
_R = 256
_CT = 512


def _emb_kernel(f_ref, w0_ref, w1_ref, o_ref):
    h = jnp.maximum(f_ref[...] * w0_ref[...], 0.0) * w1_ref[...]
    nrm = jnp.sqrt(jnp.sum(h * h, axis=1, keepdims=True))
    o_ref[...] = h / jnp.maximum(nrm, 1e-12)


_P = 4
_G = 2


def _strip_kernel(a_ref, bT_ref, o_ref, strip_ref, *, nt, n):
    a = a_ref[...]
    r = a.shape[0]
    nq = _CT // 128
    slots = [[jnp.full((r, 128), -1.0, jnp.float32) for _ in range(_P)]
             for _ in range(_G)]
    for j in range(nt):
        tile = jnp.maximum(
            jnp.dot(a, bT_ref[:, pl.ds(j * _CT, _CT)],
                    preferred_element_type=jnp.float32), 0.0)
        strip_ref[:, pl.ds(j * _CT, _CT)] = tile
        for q in range(nq):
            g = q * _G // nq
            cur = tile[:, q * 128:(q + 1) * 128]
            for s in range(_P):
                hi = jnp.maximum(slots[g][s], cur)
                cur = jnp.minimum(slots[g][s], cur)
                slots[g][s] = hi

    s8 = slots[0] + slots[1]

    def _ce(i, j):
        hi = jnp.maximum(s8[i], s8[j])
        lo = jnp.minimum(s8[i], s8[j])
        s8[i], s8[j] = hi, lo

    for i, j in [(0, 1), (2, 3), (4, 5), (6, 7),
                 (0, 2), (1, 3), (4, 6), (5, 7),
                 (1, 2), (5, 6),
                 (0, 4), (1, 5), (2, 6), (3, 7),
                 (2, 4), (3, 5),
                 (1, 2), (3, 4), (5, 6)]:
        _ce(i, j)

    t = None
    for _ in range(_KK):
        t = jnp.max(s8[0], axis=1, keepdims=True)
        shift = s8[0] == t
        for i in range(7):
            s8[i] = jnp.where(shift, s8[i + 1], s8[i])
        s8[7] = jnp.where(shift, -1.0, s8[7])

    for j in range(nt):
        start = j * _CT
        if start >= n:
            break
        w = min(_CT, n - start)
        tile = strip_ref[:, pl.ds(start, _CT)]
        masked = jnp.where(tile >= t, tile, 0.0)
        o_ref[:, pl.ds(start, w)] = masked[:, :w]


def kernel(features, w0, w1):
    n, d = features.shape
    npad = -(-n // _CT) * _CT
    nt = npad // _CT

    f = features
    if npad != n:
        f = jnp.pad(features, ((0, npad - n), (0, 0)))

    rb = 1024 if npad % 1024 == 0 else _R
    emb = pl.pallas_call(
        _emb_kernel,
        grid=(npad // rb,),
        in_specs=[
            pl.BlockSpec((rb, d), lambda i: (i, 0)),
            pl.BlockSpec((1, d), lambda i: (0, 0)),
            pl.BlockSpec((1, d), lambda i: (0, 0)),
        ],
        out_specs=pl.BlockSpec((rb, d), lambda i: (i, 0)),
        out_shape=jax.ShapeDtypeStruct((npad, d), jnp.float32),
    )(f, w0.reshape(1, d), w1.reshape(1, d))

    embT = emb.T

    import functools
    out = pl.pallas_call(
        functools.partial(_strip_kernel, nt=nt, n=n),
        grid=(npad // _R,),
        in_specs=[
            pl.BlockSpec((_R, d), lambda i: (i, 0)),
            pl.BlockSpec((d, npad), lambda i: (0, 0)),
        ],
        out_specs=pl.BlockSpec((_R, n), lambda i: (i, 0)),
        out_shape=jax.ShapeDtypeStruct((n, n), jnp.float32),
        scratch_shapes=[pltpu.VMEM((_R, npad), jnp.float32)],
        compiler_params=pltpu.CompilerParams(
            dimension_semantics=("parallel",)),
    )(emb, embT)

    return out
